# baseline (speedup 1.0000x reference)
import functools

import jax
import jax.numpy as jnp
from jax.experimental import pallas as pl
from jax.experimental.pallas import tpu as pltpu

_OBS = 8
_ACT = 4
_HID = 64
_TILE = 32768

_DN = (((0,), (0,)), ((), ()))


def _ac_kernel(x_ref, w1_ref, b1t_ref, w2_ref, b2t_ref,
               wa_ref, bat_ref, wc_ref, bct_ref,
               c0_ref, c1_ref, c2_ref, c3_ref, v_ref):
    bf16 = jnp.bfloat16
    xt = x_ref[...].astype(bf16)
    z1 = jax.lax.dot_general(
        w1_ref[...].astype(bf16), xt, _DN,
        preferred_element_type=jnp.float32)
    h1 = jnp.tanh((z1 + b1t_ref[...]).astype(bf16))
    z2 = jax.lax.dot_general(
        w2_ref[...].astype(bf16), h1, _DN, preferred_element_type=jnp.float32)
    h2 = jnp.tanh((z2 + b2t_ref[...]).astype(bf16))
    lt = jax.lax.dot_general(
        wa_ref[...].astype(bf16), h2, _DN, preferred_element_type=jnp.float32)
    lt = lt + bat_ref[...]
    vt = jax.lax.dot_general(
        wc_ref[...].astype(bf16), h2, _DN, preferred_element_type=jnp.float32)
    v_ref[...] = vt + bct_ref[...]
    c0_ref[...] = lt[0:1]
    c1_ref[...] = lt[1:2]
    c2_ref[...] = lt[2:3]
    c3_ref[...] = lt[3:4]


@functools.partial(jax.jit, static_argnames=("bp",))
def _forward(x, w1c, b1t, w2c, b2t, wa, bat, wc, bct, *, bp):
    grid = (bp // _TILE,)
    row_spec = pl.BlockSpec((1, _TILE), lambda i: (0, i))
    outs = pl.pallas_call(
        _ac_kernel,
        grid=grid,
        in_specs=[
            pl.BlockSpec((_OBS, _TILE), lambda i: (0, i)),
            pl.BlockSpec((_OBS, _HID), lambda i: (0, 0)),
            pl.BlockSpec((_HID, 1), lambda i: (0, 0)),
            pl.BlockSpec((_HID, _HID), lambda i: (0, 0)),
            pl.BlockSpec((_HID, 1), lambda i: (0, 0)),
            pl.BlockSpec((_HID, _ACT), lambda i: (0, 0)),
            pl.BlockSpec((_ACT, 1), lambda i: (0, 0)),
            pl.BlockSpec((_HID, 1), lambda i: (0, 0)),
            pl.BlockSpec((1, 1), lambda i: (0, 0)),
        ],
        out_specs=[row_spec, row_spec, row_spec, row_spec, row_spec],
        out_shape=[jax.ShapeDtypeStruct((1, bp), jnp.float32)
                   for _ in range(5)],
        compiler_params=pltpu.CompilerParams(
            dimension_semantics=("parallel",),
        ),
    )(x, w1c, b1t, w2c, b2t, wa, bat, wc, bct)
    return outs


def kernel(x, w1, b1, w2, b2, wh, bh):
    B = x.shape[0]
    bp = -(-B // _TILE) * _TILE
    if bp != B:
        x = jnp.pad(x, ((0, bp - B), (0, 0)))
    xt = x.T

    w1c = w1[:, :_HID]
    b1t = b1[:, :_HID].T
    w2c = w2[:_HID, :_HID]
    b2t = b2[:, :_HID].T
    wa = wh[:_HID, :_ACT]
    bat = bh[:, :_ACT].T
    wc = wh[:_HID, _ACT:_ACT + 1]
    bct = bh[:, _ACT:_ACT + 1]

    c0, c1, c2, c3, v = _forward(
        xt, w1c, b1t, w2c, b2t, wa, bat, wc, bct, bp=bp)
    logits = jnp.concatenate(
        [c.reshape(bp, 1) for c in (c0, c1, c2, c3)], axis=1)
    value = v.reshape(bp, 1)
    if bp != B:
        logits = logits[:B]
        value = value[:B]
    return logits, value


# speedup vs baseline: 1.4263x; 1.4263x over previous
import functools

import jax
import jax.numpy as jnp
from jax.experimental import pallas as pl
from jax.experimental.pallas import tpu as pltpu

_OBS = 8
_ACT = 4
_HID = 64
_TILE = 32768

_DN = (((0,), (0,)), ((), ()))


def _ac_kernel(x_ref, w1_ref, b1_ref, w2_ref, b2_ref, wh_ref, bh_ref,
               c0_ref, c1_ref, c2_ref, c3_ref, v_ref):
    xt = x_ref[...]
    w1c = w1_ref[...][:, :_HID]
    b1t = b1_ref[...][:, :_HID].T
    w2c = w2_ref[...][:_HID, :_HID]
    b2t = b2_ref[...][:, :_HID].T
    whc = wh_ref[...][:_HID, :]
    bht = bh_ref[...].T

    z1 = jax.lax.dot_general(
        w1c, xt, _DN, preferred_element_type=jnp.float32)
    h1 = jnp.tanh(z1 + b1t)
    z2 = jax.lax.dot_general(
        w2c, h1, _DN, preferred_element_type=jnp.float32)
    h2 = jnp.tanh(z2 + b2t)
    o = jax.lax.dot_general(
        whc, h2, _DN, preferred_element_type=jnp.float32) + bht
    c0_ref[...] = o[0:1]
    c1_ref[...] = o[1:2]
    c2_ref[...] = o[2:3]
    c3_ref[...] = o[3:4]
    v_ref[...] = o[_ACT:_ACT + 1]


@functools.partial(jax.jit, static_argnames=("bp",))
def _forward(xt, w1, b1, w2, b2, wh, bh, *, bp):
    grid = (bp // _TILE,)
    row_spec = pl.BlockSpec((1, _TILE), lambda i: (0, i))
    outs = pl.pallas_call(
        _ac_kernel,
        grid=grid,
        in_specs=[
            pl.BlockSpec((_OBS, _TILE), lambda i: (0, i)),
            pl.BlockSpec((_OBS, 128), lambda i: (0, 0)),
            pl.BlockSpec((1, 128), lambda i: (0, 0)),
            pl.BlockSpec((128, 128), lambda i: (0, 0)),
            pl.BlockSpec((1, 128), lambda i: (0, 0)),
            pl.BlockSpec((128, _OBS), lambda i: (0, 0)),
            pl.BlockSpec((1, _OBS), lambda i: (0, 0)),
        ],
        out_specs=[row_spec, row_spec, row_spec, row_spec, row_spec],
        out_shape=[jax.ShapeDtypeStruct((1, bp), jnp.float32)
                   for _ in range(5)],
        compiler_params=pltpu.CompilerParams(
            dimension_semantics=("parallel",),
        ),
    )(xt, w1, b1, w2, b2, wh, bh)
    return outs


def kernel(x, w1, b1, w2, b2, wh, bh):
    B = x.shape[0]
    bp = -(-B // _TILE) * _TILE
    if bp != B:
        x = jnp.pad(x, ((0, bp - B), (0, 0)))
    xt = x.T

    c0, c1, c2, c3, v = _forward(xt, w1, b1, w2, b2, wh, bh, bp=bp)
    logits = jnp.concatenate(
        [c.reshape(bp, 1) for c in (c0, c1, c2, c3)], axis=1)
    value = v.reshape(bp, 1)
    if bp != B:
        logits = logits[:B]
        value = value[:B]
    return logits, value


# speedup vs baseline: 1.4738x; 1.0332x over previous
import functools

import jax
import jax.numpy as jnp
from jax.experimental import pallas as pl
from jax.experimental.pallas import tpu as pltpu

_OBS = 8
_ACT = 4
_HID = 64
_TILE = 65536

_DN = (((0,), (0,)), ((), ()))


def _ac_kernel(x_ref, w1_ref, b1_ref, w2_ref, b2_ref, wh_ref, bh_ref,
               c0_ref, c1_ref, c2_ref, c3_ref, v_ref):
    xt = x_ref[...]
    w1c = w1_ref[...][:, :_HID]
    b1t = b1_ref[...][:, :_HID].T
    w2c = w2_ref[...][:_HID, :_HID]
    b2t = b2_ref[...][:, :_HID].T
    whc = wh_ref[...][:_HID, :]
    bht = bh_ref[...].T

    z1 = jax.lax.dot_general(
        w1c, xt, _DN, preferred_element_type=jnp.float32)
    h1 = jnp.tanh(z1 + b1t)
    z2 = jax.lax.dot_general(
        w2c, h1, _DN, preferred_element_type=jnp.float32)
    h2 = jnp.tanh(z2 + b2t)
    o = jax.lax.dot_general(
        whc, h2, _DN, preferred_element_type=jnp.float32) + bht
    c0_ref[...] = o[0:1]
    c1_ref[...] = o[1:2]
    c2_ref[...] = o[2:3]
    c3_ref[...] = o[3:4]
    v_ref[...] = o[_ACT:_ACT + 1]


@functools.partial(jax.jit, static_argnames=("bp",))
def _forward(xt, w1, b1, w2, b2, wh, bh, *, bp):
    grid = (bp // _TILE,)
    row_spec = pl.BlockSpec((1, _TILE), lambda i: (0, i))
    outs = pl.pallas_call(
        _ac_kernel,
        grid=grid,
        in_specs=[
            pl.BlockSpec((_OBS, _TILE), lambda i: (0, i)),
            pl.BlockSpec((_OBS, 128), lambda i: (0, 0)),
            pl.BlockSpec((1, 128), lambda i: (0, 0)),
            pl.BlockSpec((128, 128), lambda i: (0, 0)),
            pl.BlockSpec((1, 128), lambda i: (0, 0)),
            pl.BlockSpec((128, _OBS), lambda i: (0, 0)),
            pl.BlockSpec((1, _OBS), lambda i: (0, 0)),
        ],
        out_specs=[row_spec, row_spec, row_spec, row_spec, row_spec],
        out_shape=[jax.ShapeDtypeStruct((1, bp), jnp.float32)
                   for _ in range(5)],
        compiler_params=pltpu.CompilerParams(
            dimension_semantics=("parallel",),
        ),
    )(xt, w1, b1, w2, b2, wh, bh)
    return outs


def kernel(x, w1, b1, w2, b2, wh, bh):
    B = x.shape[0]
    bp = -(-B // _TILE) * _TILE
    if bp != B:
        x = jnp.pad(x, ((0, bp - B), (0, 0)))
    xt = x.T

    c0, c1, c2, c3, v = _forward(xt, w1, b1, w2, b2, wh, bh, bp=bp)
    logits = jnp.concatenate(
        [c.reshape(bp, 1) for c in (c0, c1, c2, c3)], axis=1)
    value = v.reshape(bp, 1)
    if bp != B:
        logits = logits[:B]
        value = value[:B]
    return logits, value
